# Initial kernel scaffold; baseline (speedup 1.0000x reference)
import jax
import jax.numpy as jnp
from jax.experimental import pallas as pl


def kernel(dist, angle, idx_kj, bessel_zeros, bessel_norm):
    raise NotImplementedError("write your pallas kernel here")


# speedup vs baseline: 10.6001x; 10.6001x over previous
import functools

import numpy as np
import jax
import jax.numpy as jnp
from jax import lax
from jax.experimental import pallas as pl
from jax.experimental.pallas import tpu as pltpu
from jax.experimental.pallas import tpu_sc as plsc

_NUM_SPHERICAL = 7
_NUM_RADIAL = 6
_CUTOFF = 5.0
_NK = _NUM_SPHERICAL * _NUM_RADIAL
_NKP = 48

_TB = 1024

_PREF = np.sqrt((2 * np.arange(_NUM_SPHERICAL) + 1) / (4.0 * np.pi)).astype(
    np.float32
)


def _sc_gather(dist, idx_kj):
    T = idx_kj.shape[0]
    info = plsc.get_sparse_core_info()
    nw = info.num_cores * info.num_subcores
    tpw = T // nw
    ch = 80
    nch = tpw // ch
    depth = 8

    mesh = plsc.VectorSubcoreMesh(core_axis_name="c", subcore_axis_name="s")

    @functools.partial(
        pl.kernel,
        mesh=mesh,
        out_type=jax.ShapeDtypeStruct((T,), jnp.float32),
        scratch_types=[
            pltpu.VMEM((tpw,), jnp.int32),
            pltpu.VMEM((tpw,), jnp.float32),
            pltpu.SemaphoreType.DMA,
        ],
    )
    def gather_kernel(dist_hbm, idx_hbm, out_hbm, idx_v, d_v, sem):
        wid = lax.axis_index("s") * info.num_cores + lax.axis_index("c")
        base = wid * tpw
        pltpu.sync_copy(idx_hbm.at[pl.ds(base, tpw)], idx_v)

        def chunk_copy(g):
            return pltpu.make_async_copy(
                dist_hbm.at[idx_v.at[pl.ds(g * ch, ch)]],
                d_v.at[pl.ds(g * ch, ch)],
                sem,
            )

        def fire(g, carry):
            chunk_copy(g).start()

            @pl.when(g >= depth)
            def _():
                chunk_copy(g - depth).wait()

            return carry

        lax.fori_loop(0, nch, fire, 0)

        def drain(g, carry):
            chunk_copy(nch - depth + g).wait()
            return carry

        lax.fori_loop(0, depth, drain, 0)
        pltpu.sync_copy(d_v, out_hbm.at[pl.ds(base, tpw)])

    return gather_kernel(dist, idx_kj)


def _tc_body(d_ref, a_ref, z_ref, n_ref, p_ref, out_ref):
    d = d_ref[...].reshape(1, _TB) / np.float32(_CUTOFF)
    zs = z_ref[...]

    xs = d * zs
    sin_ = jnp.sin(xs)
    cos_ = jnp.cos(xs)

    grp = lax.broadcasted_iota(jnp.int32, (_NKP, _TB), 0) // _NUM_RADIAL

    j0 = sin_ / xs
    j1 = sin_ / (xs * xs) - cos_ / xs
    res = jnp.where(grp == 0, j0, j1)
    jm, jc = j0, j1
    for i in range(1, _NUM_SPHERICAL - 1):
        jn = np.float32(2 * i + 1) / xs * jc - jm
        res = jnp.where(grp == i + 1, jn, res)
        jm, jc = jc, jn
    rbf = res * n_ref[...]

    z = jnp.cos(a_ref[...].reshape(1, _TB))
    cb = jnp.where(grp == 1, z, 1.0)
    pm = jnp.ones_like(z)
    pc = z
    for l in range(2, _NUM_SPHERICAL):
        pn = (np.float32(2 * l - 1) * z * pc - np.float32(l - 1) * pm) / np.float32(l)
        cb = jnp.where(grp == l, pn, cb)
        pm, pc = pc, pn
    cbf = cb * p_ref[...]

    out = rbf * cbf
    out_ref[...] = out.T[:, :_NK]


def _tc_math(d_g, angle, bessel_zeros, bessel_norm):
    T = d_g.shape[0]
    nblk = T // _TB

    zcol = jnp.concatenate(
        [
            bessel_zeros.reshape(_NK, 1),
            jnp.ones((_NKP - _NK, 1), jnp.float32),
        ]
    )
    ncol = jnp.concatenate(
        [
            bessel_norm.reshape(_NK, 1),
            jnp.zeros((_NKP - _NK, 1), jnp.float32),
        ]
    )
    pcol = jnp.concatenate(
        [
            jnp.asarray(np.repeat(_PREF, _NUM_RADIAL).reshape(_NK, 1)),
            jnp.zeros((_NKP - _NK, 1), jnp.float32),
        ]
    )
    d3 = d_g.reshape(nblk, 1, _TB)
    a3 = angle.reshape(nblk, 1, _TB)

    return pl.pallas_call(
        _tc_body,
        grid=(nblk,),
        in_specs=[
            pl.BlockSpec((1, 1, _TB), lambda i: (i, 0, 0)),
            pl.BlockSpec((1, 1, _TB), lambda i: (i, 0, 0)),
            pl.BlockSpec((_NKP, 1), lambda i: (0, 0)),
            pl.BlockSpec((_NKP, 1), lambda i: (0, 0)),
            pl.BlockSpec((_NKP, 1), lambda i: (0, 0)),
        ],
        out_specs=pl.BlockSpec((_TB, _NK), lambda i: (i, 0)),
        out_shape=jax.ShapeDtypeStruct((T, _NK), jnp.float32),
    )(d3, a3, zcol, ncol, pcol)


def kernel(dist, angle, idx_kj, bessel_zeros, bessel_norm):
    d_g = _sc_gather(dist, idx_kj)
    return _tc_math(d_g, angle, bessel_zeros, bessel_norm)
